# baseline (speedup 1.0000x reference)
import functools

import jax
import jax.numpy as jnp
from jax import lax
from jax.experimental import pallas as pl
from jax.experimental.pallas import tpu as pltpu
from jax.experimental.pallas import tpu_sc as plsc

FEAT_DIM = 256
NUM_FEAT = 16384
NUM_GROUPS = 4096
MAX_FEAT_PLUS = 64

NC = 2
NS = 16
L = 16
NW = NC * NS
GPW = NUM_GROUPS // NW
D = FEAT_DIM
FP = MAX_FEAT_PLUS
NB = 3

ROWS_P = NUM_FEAT // 2
SENT = ROWS_P
ROWS_STAGE = ROWS_P + 128
RPT = ROWS_STAGE // NS
RPC = 40
DW = D // 2
NCH = DW // L
CH = 16
MQ = GPW // L


def _unpack(w):
    lo = lax.bitcast_convert_type(lax.shift_left(w, jnp.int32(16)), jnp.float32)
    hi = lax.bitcast_convert_type(w & jnp.int32(-65536), jnp.float32)
    return lo, hi


def _round_bf16_bits(x):
    u = lax.bitcast_convert_type(x, jnp.int32)
    lsb = lax.shift_right_logical(u, jnp.int32(16)) & jnp.int32(1)
    return u + jnp.int32(0x7FFF) + lsb


def _pack(lo, hi):
    wl = lax.shift_right_logical(_round_bf16_bits(lo), jnp.int32(16))
    wh = _round_bf16_bits(hi) & jnp.int32(-65536)
    return wl | wh


def _grouper_body(tbl_hbm, idx_hbm, mcnt_hbm, out_hbm, spmem_tbl, idx_v,
                  mcnt_v, b0, b1, b2, stage_v, out_stage, s0, s1, s2):
    bufs = (b0, b1, b2)
    sems = (s0, s1, s2)
    cid = lax.axis_index("c")
    sid = lax.axis_index("s")
    wid = sid * NC + cid
    g0 = wid * GPW
    lanes = lax.iota(jnp.int32, L)

    def mval(g):
        return mcnt_v[pl.ds(g, L)][0]

    def run_pass(p, first):
        r0 = sid * RPT
        for c in range(RPT // RPC):
            pltpu.sync_copy(tbl_hbm.at[p, pl.ds(r0 + c * RPC, RPC)], stage_v)
            pltpu.sync_copy(stage_v, spmem_tbl.at[pl.ds(r0 + c * RPC, RPC)])
        plsc.subcore_barrier()

        pltpu.sync_copy(idx_hbm.at[p, pl.ds(g0, GPW)], idx_v)
        pltpu.sync_copy(mcnt_hbm.at[p, wid], mcnt_v)

        def issue(g, b):
            m = mval(g)

            def start_chunk(c, carry):
                pltpu.make_async_copy(
                    spmem_tbl.at[idx_v.at[g, pl.ds(CH * c, CH)]],
                    bufs[b].at[pl.ds(CH * c, CH)], sems[b]).start()
                return carry

            lax.fori_loop(0, m, start_chunk, 0)

        def drain(g, b):
            m = mval(g)

            def wait_chunk(c, carry):
                pltpu.make_async_copy(
                    spmem_tbl.at[idx_v.at[g, pl.ds(0, CH)]],
                    bufs[b].at[pl.ds(0, CH)], sems[b]).wait()
                return carry

            lax.fori_loop(0, m, wait_chunk, 0)

        for b in range(NB):
            issue(b, b)

        def process_group(g, b):
            m = mval(g)
            drain(g, b)

            def row_body(j, acc, _rows=bufs[b]):
                out = []
                for c in range(NCH):
                    lo, hi = _unpack(_rows[j, pl.ds(L * c, L)])
                    out.append(acc[2 * c] + lo)
                    out.append(acc[2 * c + 1] + hi)
                return tuple(out)

            zeros = tuple(
                jnp.zeros((L,), jnp.float32) for _ in range(2 * NCH))
            acc = lax.fori_loop(0, m * CH, row_body, zeros)
            for c in range(NCH):
                lo, hi = acc[2 * c], acc[2 * c + 1]
                if not first:
                    plo, phi = _unpack(out_stage[g, pl.ds(L * c, L)])
                    lo = lo + plo
                    hi = hi + phi
                out_stage[g, pl.ds(L * c, L)] = _pack(lo, hi)

        NFULL = GPW // NB

        def block_body(t, carry):
            for b in range(NB):
                g = t * NB + b
                process_group(g, b)
                g2 = jnp.minimum(g + NB, GPW - 1)
                issue(g2, b)
            return carry

        lax.fori_loop(0, NFULL, block_body, 0)
        for r in range(GPW % NB):
            process_group(NFULL * NB + r, r)
        for b in range(GPW % NB, NB):
            drain(GPW - 1, b)
        plsc.subcore_barrier()

    run_pass(0, True)
    run_pass(1, False)
    pltpu.sync_copy(out_stage, out_hbm.at[pl.ds(g0, GPW)])


_SCRATCH = [
    pltpu.VMEM_SHARED((ROWS_STAGE, DW), jnp.int32),
    pltpu.VMEM((GPW, FP), jnp.int32),
    pltpu.VMEM((GPW + L,), jnp.int32),
    pltpu.VMEM((FP, DW), jnp.int32),
    pltpu.VMEM((FP, DW), jnp.int32),
    pltpu.VMEM((FP, DW), jnp.int32),
    pltpu.VMEM((RPC, DW), jnp.int32),
    pltpu.VMEM((GPW, DW), jnp.int32),
    pltpu.SemaphoreType.DMA,
    pltpu.SemaphoreType.DMA,
    pltpu.SemaphoreType.DMA,
]


@functools.lru_cache(maxsize=None)
def _grouper_sc():
    mesh = plsc.VectorSubcoreMesh(
        core_axis_name="c", subcore_axis_name="s",
        num_cores=NC, num_subcores=NS)
    return pl.kernel(
        _grouper_body,
        out_type=jax.ShapeDtypeStruct((NUM_GROUPS, DW), jnp.int32),
        mesh=mesh,
        scratch_types=_SCRATCH,
    )


@jax.jit
def kernel(in_features, W, grp_edge_feat, edge_to_node, grp_edge_idx_plus,
           grp_num_feat, grp_feat_idx_plus):
    ratio = 1.0 / grp_num_feat.astype(jnp.float32)
    csum = jnp.cumsum(
        jnp.broadcast_to(ratio[:, None], (NUM_GROUPS, FP)), axis=1)
    hard = csum <= 1.0
    idx = grp_feat_idx_plus.astype(jnp.int32)
    slot = jnp.arange(FP, dtype=jnp.int32)[None, :]
    parts = []
    counts = []
    for active, local in (
        (hard & (idx < ROWS_P), idx),
        (hard & (idx >= ROWS_P), idx - ROWS_P),
    ):
        perm = jnp.argsort(jnp.where(active, slot, FP + slot), axis=1)
        parts.append(jnp.take_along_axis(
            jnp.where(active, local, SENT), perm, axis=1))
        k = jnp.sum(active.astype(jnp.int32), axis=1)
        counts.append((k + (CH - 1)) // CH)
    idx_p = jnp.stack(parts)
    mcnt = jnp.stack(counts).reshape(2, NW, GPW)
    mcnt = jnp.pad(mcnt, ((0, 0), (0, 0), (0, L)))
    tbl = in_features.astype(jnp.bfloat16)
    zpad = jnp.zeros((ROWS_STAGE - ROWS_P, D), jnp.bfloat16)
    tbl_staged = jnp.stack([
        jnp.concatenate([tbl[:ROWS_P], zpad], axis=0),
        jnp.concatenate([tbl[ROWS_P:], zpad], axis=0),
    ])
    tbl_pairs = jnp.stack(
        [tbl_staged[..., :DW], tbl_staged[..., DW:]], axis=-1)
    tbl_words = lax.bitcast_convert_type(tbl_pairs, jnp.int32)
    out_words = _grouper_sc()(tbl_words, idx_p, mcnt)
    out_pairs = lax.bitcast_convert_type(out_words, jnp.bfloat16)
    return jnp.concatenate(
        [out_pairs[..., 0], out_pairs[..., 1]], axis=-1).astype(jnp.float32)


# speedup vs baseline: 1.3721x; 1.3721x over previous
import functools

import jax
import jax.numpy as jnp
from jax import lax
from jax.experimental import pallas as pl
from jax.experimental.pallas import tpu as pltpu
from jax.experimental.pallas import tpu_sc as plsc

FEAT_DIM = 256
NUM_FEAT = 16384
NUM_GROUPS = 4096
MAX_FEAT_PLUS = 64

NC = 2
NS = 16
L = 16
NW = NC * NS
GPW = NUM_GROUPS // NW
D = FEAT_DIM
FP = MAX_FEAT_PLUS
NB = 3

ROWS_P = NUM_FEAT // 2
SENT = ROWS_P
ROWS_STAGE = ROWS_P + 128
RPT = ROWS_STAGE // NS
RPC = 40
DW = D // 2
NCH = DW // L
CH = 16
FPP = FP + CH


def _unpack(w):
    lo = lax.bitcast_convert_type(lax.shift_left(w, jnp.int32(16)), jnp.float32)
    hi = lax.bitcast_convert_type(w & jnp.int32(-65536), jnp.float32)
    return lo, hi


def _round_bf16_bits(x):
    u = lax.bitcast_convert_type(x, jnp.int32)
    lsb = lax.shift_right_logical(u, jnp.int32(16)) & jnp.int32(1)
    return u + jnp.int32(0x7FFF) + lsb


def _pack(lo, hi):
    wl = lax.shift_right_logical(_round_bf16_bits(lo), jnp.int32(16))
    wh = _round_bf16_bits(hi) & jnp.int32(-65536)
    return wl | wh


def _grouper_body(tbl_hbm, idx_hbm, mcnt_hbm, out_hbm, spmem_tbl, idx_v,
                  mcnt_v, b0, b1, b2, stage_v, out_stage, s0, s1, s2):
    bufs = (b0, b1, b2)
    sems = (s0, s1, s2)
    cid = lax.axis_index("c")
    sid = lax.axis_index("s")
    wid = sid * NC + cid
    g0 = wid * GPW
    lanes = lax.iota(jnp.int32, L)

    def mval(g, p):
        w = mcnt_v[pl.ds(g, L)][0]
        if p == 0:
            return w & jnp.int32(15)
        return lax.shift_right_logical(w, jnp.int32(4)) & jnp.int32(15)

    def run_pass(p, first):
        r0 = sid * RPT
        for c in range(RPT // RPC):
            pltpu.sync_copy(tbl_hbm.at[p, pl.ds(r0 + c * RPC, RPC)], stage_v)
            pltpu.sync_copy(stage_v, spmem_tbl.at[pl.ds(r0 + c * RPC, RPC)])
        plsc.subcore_barrier()

        if first:
            pltpu.sync_copy(idx_hbm.at[pl.ds(g0, GPW)], idx_v)
            pltpu.sync_copy(mcnt_hbm.at[pl.ds(wid * (GPW + L), GPW + L)], mcnt_v)

        def issue(g, b):
            m = mval(g, p)

            def start_chunk(c, carry):
                if p == 0:
                    src_off = CH * c
                else:
                    src_off = FPP - CH - CH * c
                pltpu.make_async_copy(
                    spmem_tbl.at[idx_v.at[g, pl.ds(src_off, CH)]],
                    bufs[b].at[pl.ds(CH * c, CH)], sems[b]).start()
                return carry

            lax.fori_loop(0, m, start_chunk, 0)

        def drain(g, b):
            m = mval(g, p)

            def wait_chunk(c, carry):
                pltpu.make_async_copy(
                    spmem_tbl.at[idx_v.at[g, pl.ds(0, CH)]],
                    bufs[b].at[pl.ds(0, CH)], sems[b]).wait()
                return carry

            lax.fori_loop(0, m, wait_chunk, 0)

        for b in range(NB):
            issue(b, b)

        def process_group(g, b):
            m = mval(g, p)
            drain(g, b)

            def row_body(j, acc, _rows=bufs[b]):
                out = []
                for c in range(NCH):
                    lo, hi = _unpack(_rows[j, pl.ds(L * c, L)])
                    out.append(acc[2 * c] + lo)
                    out.append(acc[2 * c + 1] + hi)
                return tuple(out)

            zeros = tuple(
                jnp.zeros((L,), jnp.float32) for _ in range(2 * NCH))
            acc = lax.fori_loop(0, m * CH, row_body, zeros)
            for c in range(NCH):
                lo, hi = acc[2 * c], acc[2 * c + 1]
                if not first:
                    plo, phi = _unpack(out_stage[g, pl.ds(L * c, L)])
                    lo = lo + plo
                    hi = hi + phi
                out_stage[g, pl.ds(L * c, L)] = _pack(lo, hi)

        NFULL = GPW // NB

        def block_body(t, carry):
            for b in range(NB):
                g = t * NB + b
                process_group(g, b)
                g2 = jnp.minimum(g + NB, GPW - 1)
                issue(g2, b)
            return carry

        lax.fori_loop(0, NFULL, block_body, 0)
        for r in range(GPW % NB):
            process_group(NFULL * NB + r, r)
        for b in range(GPW % NB, NB):
            drain(GPW - 1, b)
        plsc.subcore_barrier()

    run_pass(0, True)
    run_pass(1, False)
    pltpu.sync_copy(out_stage, out_hbm.at[pl.ds(g0, GPW)])


_SCRATCH = [
    pltpu.VMEM_SHARED((ROWS_STAGE, DW), jnp.int32),
    pltpu.VMEM((GPW, FPP), jnp.int32),
    pltpu.VMEM((GPW + L,), jnp.int32),
    pltpu.VMEM((FP, DW), jnp.int32),
    pltpu.VMEM((FP, DW), jnp.int32),
    pltpu.VMEM((FP, DW), jnp.int32),
    pltpu.VMEM((RPC, DW), jnp.int32),
    pltpu.VMEM((GPW, DW), jnp.int32),
    pltpu.SemaphoreType.DMA,
    pltpu.SemaphoreType.DMA,
    pltpu.SemaphoreType.DMA,
]


@functools.lru_cache(maxsize=None)
def _grouper_sc():
    mesh = plsc.VectorSubcoreMesh(
        core_axis_name="c", subcore_axis_name="s",
        num_cores=NC, num_subcores=NS)
    return pl.kernel(
        _grouper_body,
        out_type=jax.ShapeDtypeStruct((NUM_GROUPS, DW), jnp.int32),
        mesh=mesh,
        scratch_types=_SCRATCH,
    )


@jax.jit
def kernel(in_features, W, grp_edge_feat, edge_to_node, grp_edge_idx_plus,
           grp_num_feat, grp_feat_idx_plus):
    ratio = 1.0 / grp_num_feat.astype(jnp.float32)
    csum = jnp.cumsum(
        jnp.broadcast_to(ratio[:, None], (NUM_GROUPS, FP)), axis=1)
    hard = csum <= 1.0
    idx = grp_feat_idx_plus.astype(jnp.int32)
    act0 = hard & (idx < ROWS_P)
    act1 = hard & (idx >= ROWS_P)
    slot = jnp.broadcast_to(
        jnp.arange(FP, dtype=jnp.int32)[None, :], (NUM_GROUPS, FP))
    key = jnp.where(act0, slot, jnp.where(act1, 2 * FP + slot, FP + slot))
    val = jnp.where(hard, jnp.where(act0, idx, idx - ROWS_P), SENT)
    _, idx_c = lax.sort((key, val), dimension=1, num_keys=1)
    idx_c = jnp.pad(idx_c, ((0, 0), (0, CH)), constant_values=SENT)
    k0 = jnp.sum(act0.astype(jnp.int32), axis=1)
    k1 = jnp.sum(act1.astype(jnp.int32), axis=1)
    mword = ((k0 + (CH - 1)) // CH) | (((k1 + (CH - 1)) // CH) << 4)
    mcnt = jnp.pad(mword.reshape(NW, GPW), ((0, 0), (0, L))).reshape(-1)
    tbl = in_features.astype(jnp.bfloat16)
    zpad = jnp.zeros((ROWS_STAGE - ROWS_P, D), jnp.bfloat16)
    tbl_staged = jnp.stack([
        jnp.concatenate([tbl[:ROWS_P], zpad], axis=0),
        jnp.concatenate([tbl[ROWS_P:], zpad], axis=0),
    ])
    tbl_pairs = jnp.stack(
        [tbl_staged[..., :DW], tbl_staged[..., DW:]], axis=-1)
    tbl_words = lax.bitcast_convert_type(tbl_pairs, jnp.int32)
    out_words = _grouper_sc()(tbl_words, idx_c, mcnt)
    out_pairs = lax.bitcast_convert_type(out_words, jnp.bfloat16)
    return jnp.concatenate(
        [out_pairs[..., 0], out_pairs[..., 1]], axis=-1).astype(jnp.float32)
